# baseline (speedup 1.0000x reference)
import functools

import jax
import jax.numpy as jnp
from jax import lax
from jax.experimental import pallas as pl
from jax.experimental.pallas import tpu as pltpu
from jax.experimental.pallas import tpu_sc as plsc

_V = 1000000
_D = 64
_R = 8
_LANES = 16
_NDB = _D // _LANES
_NW = 32
_BPW = 128
_H = 50
_PKC = 512


def _pack_body(e_ref, a_ref, o_ref):
    et = e_ref[...].T
    at = a_ref[...].T
    o_ref[...] = jnp.concatenate([et] + [at] * 8, axis=1)


def _pack_tables(emb_t, a_t):
    grid = (_V + _PKC - 1) // _PKC
    return pl.pallas_call(
        _pack_body,
        grid=(grid,),
        in_specs=[
            pl.BlockSpec((_D, _PKC), lambda g: (0, g)),
            pl.BlockSpec((_R, _PKC), lambda g: (0, g)),
        ],
        out_specs=pl.BlockSpec((_PKC, 2 * _D), lambda g: (g, 0)),
        out_shape=jax.ShapeDtypeStruct((_V, 2 * _D), jnp.float32),
    )(emb_t, a_t)


def _make_sc_kernel():
    mesh = plsc.VectorSubcoreMesh(core_axis_name="c", subcore_axis_name="s")

    @functools.partial(
        pl.kernel,
        mesh=mesh,
        compiler_params=pltpu.CompilerParams(needs_layout_passes=False),
        out_type=jax.ShapeDtypeStruct((_H, _D // 8, _NW, 8, _BPW),
                                      jnp.float32),
        scratch_types=[
            pltpu.VMEM((_H, _BPW), jnp.int32),
            pltpu.VMEM((_BPW, 2 * _D), jnp.float32),
            pltpu.VMEM((_D // 8, 8, _BPW + 1), jnp.float32),
            pltpu.VMEM((_R, 2 * _D), jnp.float32),
            pltpu.SemaphoreType.DMA,
        ],
    )
    def sc_kernel(idx_hbm, tab_hbm, b_hbm, out_hbm,
                  idx_v, rows_v, tr_v, b_v, sem):
        num_cores = 2
        wid = lax.axis_index("s") * num_cores + lax.axis_index("c")

        pltpu.sync_copy(idx_hbm.at[:, wid], idx_v)
        pltpu.sync_copy(b_hbm, b_v)

        b_vecs = [[b_v[r, pl.ds(db * _LANES, _LANES)] for db in range(_NDB)]
                  for r in range(_R)]
        j_vecs = [lax.iota(jnp.int32, _LANES) + db * _LANES
                  for db in range(_NDB)]
        jt_vecs = [jv // 8 for jv in j_vecs]
        j8_vecs = [jv % 8 for jv in j_vecs]

        def unit_body(h, carry):
            pltpu.async_copy(tab_hbm.at[idx_v.at[h]], rows_v, sem).wait()

            def row_body(k, c):
                k_vec = jnp.full((_LANES,), k, jnp.int32)
                avs = [plsc.load_gather(
                           rows_v,
                           [k_vec, jnp.full((_LANES,), _D + r, jnp.int32)])
                       for r in range(_R)]
                for db in range(_NDB):
                    acc = rows_v[k, pl.ds(db * _LANES, _LANES)]
                    for r in range(_R):
                        acc = acc + avs[r] * b_vecs[r][db]
                    plsc.store_scatter(
                        tr_v, [jt_vecs[db], j8_vecs[db], k_vec], acc)
                return c

            lax.fori_loop(0, _BPW, row_body, 0)
            pltpu.sync_copy(tr_v.at[:, :, pl.ds(0, _BPW)],
                            out_hbm.at[h, :, wid])
            return carry

        lax.fori_loop(0, _H, unit_body, 0)

    return sc_kernel


_sc_kernel = _make_sc_kernel()


def kernel(inputs, embeddings, lora_A, lora_B):
    batch, hist = inputs.shape
    packed = _pack_tables(embeddings.T, lora_A.T)
    idx3 = inputs.T.reshape(_H, _NW, _BPW)
    b_scaled = lora_B * jnp.sqrt(jnp.asarray(_D, jnp.float32))
    b_pad = jnp.concatenate([b_scaled, b_scaled], axis=1)
    out5 = _sc_kernel(idx3, packed, b_pad)
    out = out5.transpose(2, 4, 0, 1, 3).reshape(batch, hist, _D)
    return out
